# baseline (speedup 1.0000x reference)
import functools

import jax
import jax.numpy as jnp
from jax import lax
from jax.experimental import pallas as pl
from jax.experimental.pallas import tpu as pltpu, tpu_sc as plsc

N = 10000
E = 320000
IN_C = 128
OUT_C = 64
DIM = 3
KS = 5
KT = KS ** DIM
S = 2 ** DIM
CAT_C = IN_C + 2 * OUT_C

NC = 2
NS = 16
NW = NC * NS
ET = E // NW
CE = 80
CC = CE * S
NB = CC // 128
NCHUNK = ET // CE
ZR = 200


def _basis_idx(pseudo):
    v = pseudo * (KS - 1)
    bot = jnp.floor(v)
    frac = v - bot
    bot = bot.astype(jnp.int32)
    bs, ks = [], []
    for s in range(S):
        b = jnp.ones((pseudo.shape[0],), pseudo.dtype)
        idx = jnp.zeros((pseudo.shape[0],), jnp.int32)
        stride = 1
        for d in range(DIM):
            bit = (s >> d) & 1
            b = b * (frac[:, d] if bit else (1.0 - frac[:, d]))
            kd = jnp.clip(bot[:, d] + bit, 0, KS - 1)
            idx = idx + kd * stride
            stride = stride * KS
        bs.append(b)
        ks.append(idx)
    return jnp.stack(bs, 1), jnp.stack(ks, 1)


_STORE_ORDER = [base + par + 2 * i
                for base in (0, 32) for par in (0, 1) for i in range(16)]
_INV_ORDER = [0] * OUT_C
for _s, _f in enumerate(_STORE_ORDER):
    _INV_ORDER[_f] = _s


def _sc_agg_body(with_deg, z_hbm, gidx_hbm, basis_hbm, dst_hbm, *refs):
    refs = list(refs)
    out_hbm = refs.pop(0)
    deg_hbm = refs.pop(0) if with_deg else None
    (cidx_v, basis_v, dst_v, rows_v, msg_v, zer_v, ones_v, zer16_v,
     out_sh, deg_sh, sem) = refs

    c = lax.axis_index("c")
    s = lax.axis_index("s")
    t = c * NS + s

    def zrow(r, _):
        for f in range(4):
            zer_v[r, pl.ds(f * 16, 16)] = jnp.zeros((16,), jnp.float32)
        zer16_v[r, pl.ds(0, 16)] = jnp.zeros((16,), jnp.float32)
        return 0
    lax.fori_loop(0, ZR, zrow, 0)
    for j in range(3):
        pltpu.sync_copy(zer_v, out_sh.at[pl.ds(s * 600 + j * ZR, ZR)])
    @pl.when(s == 0)
    def _():
        pltpu.sync_copy(zer_v, out_sh.at[pl.ds(9600, ZR)])
        pltpu.sync_copy(zer_v, out_sh.at[pl.ds(9800, ZR)])
    if with_deg:
        for j in range(3):
            pltpu.sync_copy(zer16_v,
                            deg_sh.at[pl.ds(s * 600 + j * ZR, ZR)])
        @pl.when(s == 0)
        def _():
            pltpu.sync_copy(zer16_v, deg_sh.at[pl.ds(9600, ZR)])
            pltpu.sync_copy(zer16_v, deg_sh.at[pl.ds(9800, ZR)])
        def o16(e, _):
            ones_v[e, pl.ds(0, 16)] = jnp.ones((16,), jnp.float32)
            return 0
        lax.fori_loop(0, CE, o16, 0)
    plsc.subcore_barrier()

    def chunk_body(ci, _):
        g0 = t * NCHUNK + ci
        e0 = t * ET + ci * CE
        p0 = t * ET * S + ci * CC
        pltpu.sync_copy(gidx_hbm.at[g0], cidx_v)
        pltpu.sync_copy(basis_hbm.at[pl.ds(p0, CC)], basis_v)
        pltpu.sync_copy(dst_hbm.at[pl.ds(e0, CE)], dst_v)
        handles = [
            pltpu.async_copy(z_hbm.at[cidx_v.at[j]],
                             rows_v.at[pl.ds(j * 128, 128)], sem)
            for j in range(NB)
        ]
        for h in handles:
            h.wait()

        def pair_body(ep, _):
            bv = basis_v[pl.ds(ep * 16, 16)]
            for half in range(2):
                e = ep * 2 + half
                base = e * S
                acc = [jnp.zeros((16,), jnp.float32) for _ in range(4)]
                for sc in range(S):
                    b = bv[half * S + sc]
                    for g in range(2):
                        iv = rows_v[base + sc, pl.ds(g * 16, 16)]
                        bfv = plsc.bitcast(iv, jnp.bfloat16)
                        lo, hi = plsc.unpack(
                            bfv, format=plsc.PackFormat.INTERLEAVED)
                        acc[2 * g] = acc[2 * g] + b * lo
                        acc[2 * g + 1] = acc[2 * g + 1] + b * hi
                for q in range(4):
                    msg_v[e, pl.ds(q * 16, 16)] = acc[q]
            return 0
        lax.fori_loop(0, CE // 2, pair_body, 0)
        pltpu.sync_copy(msg_v, out_sh.at[dst_v], add=True)
        if with_deg:
            pltpu.sync_copy(ones_v, deg_sh.at[dst_v], add=True)
        return 0
    lax.fori_loop(0, NCHUNK, chunk_body, 0)
    plsc.subcore_barrier()

    pltpu.sync_copy(out_sh.at[pl.ds(s * 600, 600)],
                    out_hbm.at[c].at[pl.ds(s * 600, 600)])
    @pl.when(s == 0)
    def _():
        pltpu.sync_copy(out_sh.at[pl.ds(9600, 400)],
                        out_hbm.at[c].at[pl.ds(9600, 400)])
    if with_deg:
        pltpu.sync_copy(deg_sh.at[pl.ds(s * 600, 600)],
                        deg_hbm.at[c].at[pl.ds(s * 600, 600)])
        @pl.when(s == 0)
        def _():
            pltpu.sync_copy(deg_sh.at[pl.ds(9600, 400)],
                            deg_hbm.at[c].at[pl.ds(9600, 400)])


def _make_sc_agg(with_deg):
    out_type = [jax.ShapeDtypeStruct((NC, N, OUT_C), jnp.float32)]
    if with_deg:
        out_type.append(jax.ShapeDtypeStruct((NC, N, 16), jnp.float32))
    scratch = [
        pltpu.VMEM((NB, 128), jnp.int32),
        pltpu.VMEM((CC,), jnp.float32),
        pltpu.VMEM((CE,), jnp.int32),
        pltpu.VMEM((CC, OUT_C // 2), jnp.int32),
        pltpu.VMEM((CE, OUT_C), jnp.float32),
        pltpu.VMEM((ZR, OUT_C), jnp.float32),
        pltpu.VMEM((CE, 16), jnp.float32),
        pltpu.VMEM((ZR, 16), jnp.float32),
        pltpu.VMEM_SHARED((N, OUT_C), jnp.float32),
        pltpu.VMEM_SHARED((N, 16), jnp.float32),
        pltpu.SemaphoreType.DMA,
    ]
    mesh = plsc.VectorSubcoreMesh(core_axis_name="c", subcore_axis_name="s")
    return pl.kernel(
        functools.partial(_sc_agg_body, with_deg),
        out_type=tuple(out_type),
        mesh=mesh,
        scratch_types=scratch,
        compiler_params=pltpu.CompilerParams(use_tc_tiling_on_sc=False,
                                             needs_layout_passes=False),
    )


_sc_agg_deg = _make_sc_agg(True)
_sc_agg = _make_sc_agg(False)


def _final_mm_body(cat_ref, wf_ref, bf_ref, o_ref):
    o_ref[...] = jnp.dot(cat_ref[...], wf_ref[...],
                         preferred_element_type=jnp.float32) + bf_ref[...]


def _final_mm(cat, wf, bf):
    nb = 1000
    return pl.pallas_call(
        _final_mm_body,
        grid=(N // nb,),
        in_specs=[
            pl.BlockSpec((nb, CAT_C), lambda i: (i, 0)),
            pl.BlockSpec((CAT_C, OUT_C), lambda i: (0, 0)),
            pl.BlockSpec((1, OUT_C), lambda i: (0, 0)),
        ],
        out_specs=pl.BlockSpec((nb, OUT_C), lambda i: (i, 0)),
        out_shape=jax.ShapeDtypeStruct((N, OUT_C), jnp.float32),
    )(cat, wf, bf.reshape(1, OUT_C))


def kernel(x, edge_index, edge_attr, w1, r1, b1, w2, r2, b2, wf, bf):
    src = edge_index[0]
    dst = edge_index[1]
    basis, kidx = _basis_idx(edge_attr)
    gidx = (src[:, None] * KT + kidx).reshape(NW * NCHUNK, NB, 128)
    basis_flat = basis.reshape(E * S)

    def layer(xin, w, root, bias, deginv, with_deg):
        in_c = xin.shape[1]
        wflat = w.transpose(1, 0, 2).reshape(in_c, KT * OUT_C)
        zb = jnp.dot(xin.astype(jnp.bfloat16), wflat.astype(jnp.bfloat16),
                     preferred_element_type=jnp.bfloat16)
        z = lax.bitcast_convert_type(
            zb.reshape(N * KT, OUT_C // 2, 2), jnp.int32)
        if with_deg:
            outp, degp = _sc_agg_deg(z, gidx, basis_flat, dst)
            deg = degp[0, :, 0] + degp[1, :, 0]
            deginv = 1.0 / jnp.clip(deg, 1.0, None)
        else:
            (outp,) = _sc_agg(z, gidx, basis_flat, dst)
        agg = (outp[0] + outp[1])[:, jnp.array(_INV_ORDER)]
        h = jax.nn.relu(agg * deginv[:, None] + xin @ root + bias)
        return h, deginv

    h1, deginv = layer(x, w1, r1, b1, None, True)
    h2, _ = layer(h1, w2, r2, b2, deginv, False)
    cat = jnp.concatenate([x, h1, h2], axis=-1)
    return _final_mm(cat, wf, bf)


# speedup vs baseline: 46.3344x; 46.3344x over previous
import functools

import jax
import jax.numpy as jnp
from jax import lax
from jax.experimental import pallas as pl
from jax.experimental.pallas import tpu as pltpu, tpu_sc as plsc

N = 10000
E = 320000
IN_C = 128
OUT_C = 64
DIM = 3
KS = 5
KT = KS ** DIM
S = 2 ** DIM
CAT_C = IN_C + 2 * OUT_C

NC = 2
NS = 16
NW = NC * NS
ET = E // NW
CE = 40
CC = CE * S
GB = 64
NB = CC // GB
NCHUNK = ET // CE
ZR = 40


def _basis_idx(pseudo):
    v = pseudo * (KS - 1)
    bot = jnp.floor(v)
    frac = v - bot
    bot = bot.astype(jnp.int32)
    bs, ks = [], []
    for s in range(S):
        b = jnp.ones((pseudo.shape[0],), pseudo.dtype)
        idx = jnp.zeros((pseudo.shape[0],), jnp.int32)
        stride = 1
        for d in range(DIM):
            bit = (s >> d) & 1
            b = b * (frac[:, d] if bit else (1.0 - frac[:, d]))
            kd = jnp.clip(bot[:, d] + bit, 0, KS - 1)
            idx = idx + kd * stride
            stride = stride * KS
        bs.append(b)
        ks.append(idx)
    return jnp.stack(bs, 1), jnp.stack(ks, 1)


def _sc_agg_body(with_deg, z_hbm, gidx_hbm, basis_hbm, dst_hbm, *refs):
    refs = list(refs)
    out_hbm = refs.pop(0)
    deg_hbm = refs.pop(0) if with_deg else None
    (cidx_v, basis_v, dst_v, rows_v, cidx_w, basis_w, dst_w, rows_w,
     msg_v, zer_v, ones_v, zer16_v, out_sh, deg_sh,
     sem_in0, sem_g0, sem_in1, sem_g1, sem) = refs

    c = lax.axis_index("c")
    s = lax.axis_index("s")
    t = c * NS + s

    def zrow(r, _):
        for f in range(4):
            zer_v[r, pl.ds(f * 16, 16)] = jnp.zeros((16,), jnp.float32)
        zer16_v[r, pl.ds(0, 16)] = jnp.zeros((16,), jnp.float32)
        return 0
    lax.fori_loop(0, ZR, zrow, 0)
    for j in range(600 // ZR):
        pltpu.sync_copy(zer_v, out_sh.at[pl.ds(s * 600 + j * ZR, ZR)])
    @pl.when(s == 0)
    def _():
        for j in range(400 // ZR):
            pltpu.sync_copy(zer_v, out_sh.at[pl.ds(9600 + j * ZR, ZR)])
    if with_deg:
        for j in range(600 // ZR):
            pltpu.sync_copy(zer16_v,
                            deg_sh.at[pl.ds(s * 600 + j * ZR, ZR)])
        @pl.when(s == 0)
        def _():
            for j in range(400 // ZR):
                pltpu.sync_copy(zer16_v, deg_sh.at[pl.ds(9600 + j * ZR, ZR)])
        def o16(e, _):
            ones_v[e, pl.ds(0, 16)] = jnp.ones((16,), jnp.float32)
            return 0
        lax.fori_loop(0, CE, o16, 0)
    plsc.subcore_barrier()

    bufs = ((cidx_v, basis_v, dst_v, rows_v, sem_in0, sem_g0),
            (cidx_w, basis_w, dst_w, rows_w, sem_in1, sem_g1))

    def fire_inputs(ci, b):
        cidx, basis, dstb, _, sem_i, _ = b
        g0 = t * NCHUNK + ci
        e0 = t * ET + ci * CE
        p0 = t * ET * S + ci * CC
        pltpu.async_copy(gidx_hbm.at[g0], cidx, sem_i)
        pltpu.async_copy(basis_hbm.at[pl.ds(p0, CC)], basis, sem_i)
        pltpu.async_copy(dst_hbm.at[pl.ds(e0, CE)], dstb, sem_i)

    def wait_inputs(b):
        cidx, basis, dstb, _, sem_i, _ = b
        pltpu.make_async_copy(gidx_hbm.at[0], cidx, sem_i).wait()
        pltpu.make_async_copy(basis_hbm.at[pl.ds(0, CC)], basis, sem_i).wait()
        pltpu.make_async_copy(dst_hbm.at[pl.ds(0, CE)], dstb, sem_i).wait()

    def fire_gathers(b):
        cidx, _, _, rows, _, sem_g = b
        for j in range(NB):
            pltpu.async_copy(z_hbm.at[cidx.at[j]],
                             rows.at[pl.ds(j * GB, GB)], sem_g)

    def wait_gathers(b):
        cidx, _, _, rows, _, sem_g = b
        for j in range(NB):
            pltpu.make_async_copy(z_hbm.at[cidx.at[j]],
                                  rows.at[pl.ds(j * GB, GB)], sem_g).wait()

    def compute_scatter(b):
        _, basis, dstb, rows, _, _ = b

        def pair_body(ep, _):
            bv = basis[pl.ds(ep * 16, 16)]
            for half in range(2):
                e = ep * 2 + half
                base = e * S
                acc = [jnp.zeros((16,), jnp.float32) for _ in range(4)]
                for sc in range(S):
                    bb = bv[half * S + sc]
                    for q in range(4):
                        acc[q] = acc[q] + bb * rows[base + sc,
                                                    pl.ds(q * 16, 16)]
                for q in range(4):
                    msg_v[e, pl.ds(q * 16, 16)] = acc[q]
            return 0
        lax.fori_loop(0, CE // 2, pair_body, 0)
        pltpu.sync_copy(msg_v, out_sh.at[dstb], add=True)
        if with_deg:
            pltpu.sync_copy(ones_v, deg_sh.at[dstb], add=True)

    fire_inputs(0, bufs[0])
    wait_inputs(bufs[0])
    fire_gathers(bufs[0])
    fire_inputs(1, bufs[1])

    def pipe_body(i, _):
        for par in range(2):
            @pl.when((i % 2) == par)
            def _():
                bp, bq = bufs[par], bufs[1 - par]
                wait_inputs(bq)
                fire_gathers(bq)
                wait_gathers(bp)
                compute_scatter(bp)
                @pl.when(i + 2 < NCHUNK)
                def _():
                    fire_inputs(i + 2, bp)
        return 0
    lax.fori_loop(0, NCHUNK - 1, pipe_body, 0)
    wait_gathers(bufs[(NCHUNK - 1) % 2])
    compute_scatter(bufs[(NCHUNK - 1) % 2])
    plsc.subcore_barrier()

    pltpu.sync_copy(out_sh.at[pl.ds(s * 600, 600)],
                    out_hbm.at[c].at[pl.ds(s * 600, 600)])
    @pl.when(s == 0)
    def _():
        pltpu.sync_copy(out_sh.at[pl.ds(9600, 400)],
                        out_hbm.at[c].at[pl.ds(9600, 400)])
    if with_deg:
        pltpu.sync_copy(deg_sh.at[pl.ds(s * 600, 600)],
                        deg_hbm.at[c].at[pl.ds(s * 600, 600)])
        @pl.when(s == 0)
        def _():
            pltpu.sync_copy(deg_sh.at[pl.ds(9600, 400)],
                            deg_hbm.at[c].at[pl.ds(9600, 400)])


def _make_sc_agg(with_deg):
    out_type = [jax.ShapeDtypeStruct((NC, N, OUT_C), jnp.float32)]
    if with_deg:
        out_type.append(jax.ShapeDtypeStruct((NC, N, 16), jnp.float32))
    scratch = [
        pltpu.VMEM((NB, GB), jnp.int32),
        pltpu.VMEM((CC,), jnp.float32),
        pltpu.VMEM((CE,), jnp.int32),
        pltpu.VMEM((CC, OUT_C), jnp.float32),
        pltpu.VMEM((NB, GB), jnp.int32),
        pltpu.VMEM((CC,), jnp.float32),
        pltpu.VMEM((CE,), jnp.int32),
        pltpu.VMEM((CC, OUT_C), jnp.float32),
        pltpu.VMEM((CE, OUT_C), jnp.float32),
        pltpu.VMEM((ZR, OUT_C), jnp.float32),
        pltpu.VMEM((CE, 16), jnp.float32),
        pltpu.VMEM((ZR, 16), jnp.float32),
        pltpu.VMEM_SHARED((N, OUT_C), jnp.float32),
        pltpu.VMEM_SHARED((N, 16), jnp.float32),
        pltpu.SemaphoreType.DMA,
        pltpu.SemaphoreType.DMA,
        pltpu.SemaphoreType.DMA,
        pltpu.SemaphoreType.DMA,
        pltpu.SemaphoreType.DMA,
    ]
    mesh = plsc.VectorSubcoreMesh(core_axis_name="c", subcore_axis_name="s")
    return pl.kernel(
        functools.partial(_sc_agg_body, with_deg),
        out_type=tuple(out_type),
        mesh=mesh,
        scratch_types=scratch,
        compiler_params=pltpu.CompilerParams(use_tc_tiling_on_sc=False),
    )


_sc_agg_deg = _make_sc_agg(True)
_sc_agg = _make_sc_agg(False)


def _final_mm_body(cat_ref, wf_ref, bf_ref, o_ref):
    o_ref[...] = jnp.dot(cat_ref[...], wf_ref[...],
                         preferred_element_type=jnp.float32) + bf_ref[...]


def _final_mm(cat, wf, bf):
    nb = 1000
    return pl.pallas_call(
        _final_mm_body,
        grid=(N // nb,),
        in_specs=[
            pl.BlockSpec((nb, CAT_C), lambda i: (i, 0)),
            pl.BlockSpec((CAT_C, OUT_C), lambda i: (0, 0)),
            pl.BlockSpec((1, OUT_C), lambda i: (0, 0)),
        ],
        out_specs=pl.BlockSpec((nb, OUT_C), lambda i: (i, 0)),
        out_shape=jax.ShapeDtypeStruct((N, OUT_C), jnp.float32),
    )(cat, wf, bf.reshape(1, OUT_C))


def kernel(x, edge_index, edge_attr, w1, r1, b1, w2, r2, b2, wf, bf):
    src = edge_index[0]
    dst = edge_index[1]
    basis, kidx = _basis_idx(edge_attr)
    gidx = (src[:, None] * KT + kidx).reshape(NW * NCHUNK, NB, GB)
    basis_flat = basis.reshape(E * S)

    def layer(xin, w, root, bias, deginv, with_deg):
        in_c = xin.shape[1]
        wflat = w.transpose(1, 0, 2).reshape(in_c, KT * OUT_C)
        z = jnp.dot(xin.astype(jnp.bfloat16), wflat.astype(jnp.bfloat16),
                    preferred_element_type=jnp.float32).reshape(N * KT, OUT_C)
        if with_deg:
            outp, degp = _sc_agg_deg(z, gidx, basis_flat, dst)
            deg = degp[0, :, 0] + degp[1, :, 0]
            deginv = 1.0 / jnp.clip(deg, 1.0, None)
        else:
            (outp,) = _sc_agg(z, gidx, basis_flat, dst)
        agg = outp[0] + outp[1]
        h = jax.nn.relu(agg * deginv[:, None] + xin @ root + bias)
        return h, deginv

    h1, deginv = layer(x, w1, r1, b1, None, True)
    h2, _ = layer(h1, w2, r2, b2, deginv, False)
    cat = jnp.concatenate([x, h1, h2], axis=-1)
    return _final_mm(cat, wf, bf)
